# baseline (speedup 1.0000x reference)
import functools

import jax
import jax.numpy as jnp
from jax import lax
from jax.experimental import pallas as pl
from jax.experimental.pallas import tpu as pltpu
from jax.experimental.pallas import tpu_sc as plsc

N_CODES = 8192
DIM = 32
N_VECS = 8192
BN = 1024
CK = 2048
BETA = 0.25


def _tc_body(zsq_ref, cbsq_ref, z_ref, cb_ref, ids_ref, loss_ref):
    zbf = z_ref[...]
    zsq = zsq_ref[...]
    run_min = jnp.full((BN, 1), jnp.inf, dtype=jnp.float32)
    run_arg = jnp.zeros((BN, 1), dtype=jnp.int32)
    true_min = jnp.full((BN, 1), jnp.inf, dtype=jnp.float32)
    col = lax.broadcasted_iota(jnp.int32, (BN, CK), 1).astype(jnp.float32)
    for c in range(N_CODES // CK):
        cbc = cb_ref[c * CK:(c + 1) * CK, :]
        cbsq = cbsq_ref[:, c * CK:(c + 1) * CK]
        mm = lax.dot_general(zbf, cbc,
                             (((1,), (1,)), ((), ())),
                             preferred_element_type=jnp.float32)
        dist = (zsq + mm) + cbsq
        mval = jnp.min(dist, axis=1, keepdims=True)
        marg_f = jnp.min(jnp.where(dist == mval, col, jnp.float32(2**30)),
                         axis=1, keepdims=True)
        marg = marg_f.astype(jnp.int32)
        upd = mval < run_min
        run_arg = jnp.where(upd, marg + jnp.int32(c * CK), run_arg)
        run_min = jnp.where(
            upd, mval.astype(jnp.bfloat16).astype(jnp.float32), run_min)
        true_min = jnp.minimum(true_min, mval)
    ids_ref[...] = run_arg
    part = jnp.sum(true_min, axis=(0, 1), keepdims=True)
    i = pl.program_id(0)

    @pl.when(i == 0)
    def _init():
        loss_ref[...] = part

    @pl.when(i != 0)
    def _acc():
        loss_ref[...] += part


def _tc_argmin(zsq, cbsq, z_f, cb):
    n = z_f.shape[0]
    return pl.pallas_call(
        _tc_body,
        grid=(n // BN,),
        in_specs=[
            pl.BlockSpec((BN, 1), lambda i: (i, 0)),
            pl.BlockSpec((1, N_CODES), lambda i: (0, 0)),
            pl.BlockSpec((BN, DIM), lambda i: (i, 0)),
            pl.BlockSpec((N_CODES, DIM), lambda i: (0, 0)),
        ],
        out_specs=[
            pl.BlockSpec((BN, 1), lambda i: (i, 0)),
            pl.BlockSpec((1, 1), lambda i: (0, 0)),
        ],
        out_shape=[
            jax.ShapeDtypeStruct((n, 1), jnp.int32),
            jax.ShapeDtypeStruct((1, 1), jnp.float32),
        ],
    )(zsq, cbsq, z_f, cb)


_NC = 2
_NS = 16
_NW = _NC * _NS
_BPW = N_VECS // _NW
_GCHUNK = 128
_DPAD = 128
_NHALVES = 1


@functools.lru_cache(maxsize=None)
def _make_sc_gather(nrows):
    bpw = nrows // _NW
    gchunk = min(bpw, _GCHUNK)

    @functools.partial(
        pl.kernel,
        mesh=plsc.VectorSubcoreMesh(core_axis_name="c", subcore_axis_name="s"),
        out_type=jax.ShapeDtypeStruct((nrows, _DPAD), jnp.float32),
        scratch_types=[
            pltpu.VMEM((gchunk,), jnp.int32),
            pltpu.VMEM((gchunk, _DPAD), jnp.float32),
            pltpu.SemaphoreType.DMA,
        ],
    )
    def _sc_gather(cb_hbm, ids_hbm, out_hbm, idx_v, rows_v, sem):
        wid = lax.axis_index("s") * _NC + lax.axis_index("c")
        base = wid * bpw
        for j in range(bpw // gchunk):
            off = base + j * gchunk
            pltpu.sync_copy(ids_hbm.at[pl.ds(off, gchunk)], idx_v)
            pltpu.async_copy(cb_hbm.at[idx_v], rows_v, sem).wait()
            pltpu.sync_copy(rows_v, out_hbm.at[pl.ds(off, gchunk)])

    return _sc_gather


def kernel(z, codebook):
    z_f = z.reshape(-1, DIM)
    zsq = jnp.sum(z ** 2, axis=-1).reshape(-1, 1)
    cbsq = jnp.sum(codebook ** 2, axis=-1)[None, :]
    cbm2 = -2.0 * codebook
    cb_pad = jnp.pad(codebook, ((0, 0), (0, _DPAD - DIM)))
    nh = N_VECS // _NHALVES
    gather = _make_sc_gather(nh)
    ids_parts, zq_parts, lacc = [], [], None
    zbf = z_f.astype(jnp.bfloat16)
    for h in range(_NHALVES):
        sl = slice(h * nh, (h + 1) * nh)
        ids2, lh = _tc_argmin(zsq[sl], cbsq, zbf[sl], cbm2)
        ids_h = ids2.reshape(-1)
        ids_parts.append(ids_h)
        zq_parts.append(gather(cb_pad, ids_h))
        lacc = lh if lacc is None else lacc + lh
    ids = jnp.concatenate(ids_parts)
    z_q = jnp.concatenate(zq_parts)[:, :DIM].reshape(z.shape)
    loss = lacc[0, 0] * jnp.float32((1.0 + BETA) / (N_VECS * DIM))
    return (z, z_q, loss, ids)


# speedup vs baseline: 1.0007x; 1.0007x over previous
import functools

import jax
import jax.numpy as jnp
from jax import lax
from jax.experimental import pallas as pl
from jax.experimental.pallas import tpu as pltpu
from jax.experimental.pallas import tpu_sc as plsc

N_CODES = 8192
DIM = 32
N_VECS = 8192
BN = 1024
CK = 2048
BETA = 0.25


def _tc_body(zsq_ref, cbsq_ref, z_ref, cb_ref, ids_ref, loss_ref):
    zbf = z_ref[...]
    zsq = zsq_ref[...]
    run_min = jnp.full((BN, 1), jnp.inf, dtype=jnp.float32)
    run_arg = jnp.zeros((BN, 1), dtype=jnp.int32)
    true_min = jnp.full((BN, 1), jnp.inf, dtype=jnp.float32)
    col = lax.broadcasted_iota(jnp.int32, (BN, CK), 1).astype(jnp.float32)
    for c in range(N_CODES // CK):
        cbc = cb_ref[c * CK:(c + 1) * CK, :]
        cbsq = cbsq_ref[:, c * CK:(c + 1) * CK]
        mm = lax.dot_general(zbf, cbc,
                             (((1,), (1,)), ((), ())),
                             preferred_element_type=jnp.float32)
        dist = (zsq + mm) + cbsq
        mval = jnp.min(dist, axis=1, keepdims=True)
        marg_f = jnp.min(jnp.where(dist == mval, col, jnp.float32(2**30)),
                         axis=1, keepdims=True)
        marg = marg_f.astype(jnp.int32)
        upd = mval < run_min
        run_arg = jnp.where(upd, marg + jnp.int32(c * CK), run_arg)
        run_min = jnp.where(
            upd, mval.astype(jnp.bfloat16).astype(jnp.float32), run_min)
        true_min = jnp.minimum(true_min, mval)
    ids_ref[...] = run_arg
    part = jnp.sum(true_min, axis=(0, 1), keepdims=True)
    i = pl.program_id(0)

    @pl.when(i == 0)
    def _init():
        loss_ref[...] = part

    @pl.when(i != 0)
    def _acc():
        loss_ref[...] += part


def _tc_argmin(zsq, cbsq, z_f, cb):
    n = z_f.shape[0]
    return pl.pallas_call(
        _tc_body,
        grid=(n // BN,),
        in_specs=[
            pl.BlockSpec((BN, 1), lambda i: (i, 0)),
            pl.BlockSpec((1, N_CODES), lambda i: (0, 0)),
            pl.BlockSpec((BN, DIM), lambda i: (i, 0)),
            pl.BlockSpec((N_CODES, DIM), lambda i: (0, 0)),
        ],
        out_specs=[
            pl.BlockSpec((BN, 1), lambda i: (i, 0)),
            pl.BlockSpec((1, 1), lambda i: (0, 0)),
        ],
        out_shape=[
            jax.ShapeDtypeStruct((n, 1), jnp.int32),
            jax.ShapeDtypeStruct((1, 1), jnp.float32),
        ],
    )(zsq, cbsq, z_f, cb)


_NC = 2
_NS = 16
_NW = _NC * _NS
_BPW = N_VECS // _NW
_GCHUNK = 128
_DPAD = 128
_NHALVES = 1


@functools.lru_cache(maxsize=None)
def _make_sc_gather(nrows):
    bpw = nrows // _NW
    gchunk = min(bpw, _GCHUNK)
    nch = bpw // gchunk

    @functools.partial(
        pl.kernel,
        mesh=plsc.VectorSubcoreMesh(core_axis_name="c", subcore_axis_name="s"),
        out_type=jax.ShapeDtypeStruct((nrows, _DPAD), jnp.float32),
        scratch_types=[
            pltpu.VMEM((bpw,), jnp.int32),
            pltpu.VMEM((bpw, _DPAD), jnp.float32),
            pltpu.SemaphoreType.DMA,
        ],
    )
    def _sc_gather(cb_hbm, ids_hbm, out_hbm, idx_v, rows_v, sem):
        wid = lax.axis_index("s") * _NC + lax.axis_index("c")
        base = wid * bpw
        pltpu.sync_copy(ids_hbm.at[pl.ds(base, bpw)], idx_v)
        copies = [
            pltpu.async_copy(cb_hbm.at[idx_v.at[pl.ds(j * gchunk, gchunk)]],
                             rows_v.at[pl.ds(j * gchunk, gchunk)], sem)
            for j in range(nch)
        ]
        for cp in copies:
            cp.wait()
        pltpu.sync_copy(rows_v, out_hbm.at[pl.ds(base, bpw)])

    return _sc_gather


def kernel(z, codebook):
    z_f = z.reshape(-1, DIM)
    zsq = jnp.sum(z ** 2, axis=-1).reshape(-1, 1)
    cbsq = jnp.sum(codebook ** 2, axis=-1)[None, :]
    cbm2 = -2.0 * codebook
    cb_pad = jnp.pad(codebook, ((0, 0), (0, _DPAD - DIM)))
    nh = N_VECS // _NHALVES
    gather = _make_sc_gather(nh)
    ids_parts, zq_parts, lacc = [], [], None
    zbf = z_f.astype(jnp.bfloat16)
    for h in range(_NHALVES):
        sl = slice(h * nh, (h + 1) * nh)
        ids2, lh = _tc_argmin(zsq[sl], cbsq, zbf[sl], cbm2)
        ids_h = ids2.reshape(-1)
        ids_parts.append(ids_h)
        zq_parts.append(gather(cb_pad, ids_h))
        lacc = lh if lacc is None else lacc + lh
    ids = jnp.concatenate(ids_parts)
    z_q = jnp.concatenate(zq_parts)[:, :DIM].reshape(z.shape)
    loss = lacc[0, 0] * jnp.float32((1.0 + BETA) / (N_VECS * DIM))
    return (z, z_q, loss, ids)
